# baseline (speedup 1.0000x reference)
import jax
import jax.numpy as jnp
from jax import lax
from jax.experimental import pallas as pl
from jax.experimental.pallas import tpu as pltpu
from jax.experimental.pallas import tpu_sc as plsc

N, H, W, K, F, D = 8, 224, 224, 1, 100000, 3
HW = H * W
P = N * HW
NW = 32
PW = P // NW
CHUNK = 1792
NCHUNK = PW // CHUNK
NVEC = CHUNK // 16
NREC = N * F
RPW = NREC // NW
RCH = 1000
NRCH = RPW // RCH
NSTR = 7
SLEN = 256


def _detile_body(tv_hbm, tr_hbm, inbuf, outbuf):
    wid = lax.axis_index("c") * 16 + lax.axis_index("s")
    iota = lax.iota(jnp.int32, 16)
    for c in range(NRCH):
        base = wid * RPW + c * RCH
        pltpu.sync_copy(tv_hbm.at[:, pl.ds(base, RCH)],
                        inbuf.at[:, pl.ds(0, RCH)])
        def sc_body(j, _):
            rec = 16 * j + iota
            for p in range(9):
                plsc.store_scatter(outbuf, [rec, jnp.full((16,), p, jnp.int32)],
                                   inbuf[p, pl.ds(16 * j, 16)])
            return 0
        lax.fori_loop(0, 63, sc_body, 0)
        pltpu.sync_copy(outbuf.at[pl.ds(0, RCH)],
                        tr_hbm.at[pl.ds(base, RCH)])


def _raster_body(attr_hbm, bary_hbm, p2f_hbm, out_hbm,
                 idx_raw, idx2, rowix, bary_v, rows_v, acc_v, sem0, sem1):
    wid = lax.axis_index("c") * 16 + lax.axis_index("s")
    img = wid // 4
    quad = wid - img * 4
    iota = lax.iota(jnp.int32, 16)
    sems = (sem0, sem1)

    def zero_body(i, _):
        idx2[0, pl.ds(i * 16, 16)] = jnp.zeros((16,), jnp.int32)
        idx2[1, pl.ds(i * 16, 16)] = jnp.zeros((16,), jnp.int32)
        return 0
    lax.fori_loop(0, (CHUNK + 16) // 16, zero_body, 0)

    def stage(c, buf):
        pbase = wid * PW + c * CHUNK
        pltpu.sync_copy(p2f_hbm.at[pl.ds(pbase, CHUNK)], idx_raw.at[buf])

        def idx_body(i, off):
            raw = idx_raw[buf, pl.ds(i * 16, 16)]
            m = raw > -1
            mi = m.astype(jnp.int32)
            cs = plsc.cumsum(mi)
            rowix[buf, pl.ds(i * 16, 16)] = jnp.maximum(off + cs - 1, 0)
            plsc.store_compressed(idx2.at[buf, pl.ds(off, 16)],
                                  jnp.maximum(raw, 0), mask=m)
            return off + jnp.sum(mi)
        nvis = lax.fori_loop(0, NVEC, idx_body, 0)

        nstr = (nvis + (SLEN - 1)) >> 8
        for s in range(NSTR):
            @pl.when(s < nstr)
            def _(s=s):
                pltpu.async_copy(
                    attr_hbm.at[idx2.at[buf, pl.ds(s * SLEN, SLEN)]],
                    rows_v.at[buf, pl.ds(s * SLEN, SLEN)], sems[buf])
        pltpu.sync_copy(bary_hbm.at[pl.ds(wid * 56 + c * 8, 8), :],
                        bary_v.at[buf])
        return nstr

    def drain(buf, nstr):
        for s in range(NSTR):
            @pl.when(s < nstr)
            def _(s=s):
                pltpu.make_async_copy(
                    attr_hbm.at[idx2.at[buf, pl.ds(s * SLEN, SLEN)]],
                    rows_v.at[buf, pl.ds(s * SLEN, SLEN)], sems[buf]).wait()

    nstr_cur = stage(0, 0)
    for c in range(NCHUNK):
        buf = c & 1
        nstr_nxt = stage(c + 1, 1 - buf) if c + 1 < NCHUNK else None
        drain(buf, nstr_cur)

        def row_body(g, _):
            def vec_body(j, _):
                pix = g * 224 + 16 * j
                raw = idx_raw[buf, pl.ds(pix, 16)]
                vis = jnp.where(raw < 0, 0.0, 1.0)
                rv = rowix[buf, pl.ds(pix, 16)]
                r = [plsc.load_gather(
                        rows_v.at[buf],
                        [rv, jnp.full((16,), cc, jnp.int32)])
                     for cc in range(9)]
                b = [bary_v[buf, g, pl.ds(v * 224 + 16 * j, 16)]
                     for v in range(3)]
                for d in range(3):
                    acc_v[d, pl.ds(pix, 16)] = vis * (b[0] * r[d]
                                                      + b[1] * r[3 + d]
                                                      + b[2] * r[6 + d])
                acc_v[3, pl.ds(pix, 16)] = vis
                return 0
            lax.fori_loop(0, 14, vec_body, 0)
            return 0
        lax.fori_loop(0, 8, row_body, 0)

        obase = img * (4 * HW) + quad * PW + c * CHUNK
        for ch in range(4):
            pltpu.sync_copy(acc_v.at[ch], out_hbm.at[pl.ds(obase + ch * HW,
                                                           CHUNK)])
        nstr_cur = nstr_nxt


def _sc_mesh():
    return plsc.VectorSubcoreMesh(core_axis_name="c", subcore_axis_name="s")


_CP = pltpu.CompilerParams(use_tc_tiling_on_sc=False,
                           needs_layout_passes=False)


@jax.jit
def _run(attrs, bary_coords, pix_to_face):
    tv = jnp.transpose(attrs, (2, 3, 0, 1)).reshape(9, NREC)
    baryn = jnp.transpose(bary_coords, (0, 1, 4, 3, 2)).reshape(N * H, 3 * W)
    p2f = pix_to_face.reshape(P)

    tr = pl.kernel(
        _detile_body,
        out_type=jax.ShapeDtypeStruct((NREC, 16), jnp.float32),
        mesh=_sc_mesh(),
        compiler_params=_CP,
        scratch_types=[
            pltpu.VMEM((9, RCH + 8), jnp.float32),
            pltpu.VMEM((RCH + 8, 16), jnp.float32),
        ],
    )(tv)

    out = pl.kernel(
        _raster_body,
        out_type=jax.ShapeDtypeStruct((N * 4 * HW,), jnp.float32),
        mesh=_sc_mesh(),
        compiler_params=_CP,
        scratch_types=[
            pltpu.VMEM((2, CHUNK), jnp.int32),
            pltpu.VMEM((2, CHUNK + 16), jnp.int32),
            pltpu.VMEM((2, CHUNK), jnp.int32),
            pltpu.VMEM((2, 8, 3 * W), jnp.float32),
            pltpu.VMEM((2, CHUNK, 16), jnp.float32),
            pltpu.VMEM((4, CHUNK), jnp.float32),
            pltpu.SemaphoreType.DMA,
            pltpu.SemaphoreType.DMA,
        ],
    )(tr, baryn, p2f)
    return out.reshape(N, 4, H, W)


def kernel(attributes, bary_coords, pix_to_face):
    return _run(attributes, bary_coords, pix_to_face)
